# Initial kernel scaffold; baseline (speedup 1.0000x reference)
import functools

import jax
import jax.numpy as jnp
from jax import lax
from jax.experimental import pallas as pl
from jax.experimental.pallas import tpu as pltpu
from jax.experimental.pallas import tpu_sc as plsc

_N = 10000
_E = 320000
_D = 128

_TILES = 32
_CH = 128
_NCHUNK = 80
_EPT = _CH * _NCHUNK
_EPAD = _TILES * _EPT
_RPS = _N // 16
_ZR = 125


def _matmul(x, W):
    def mm(x_ref, w_ref, o_ref):
        o_ref[...] = jnp.dot(x_ref[...], w_ref[...],
                             preferred_element_type=jnp.float32)

    return pl.pallas_call(
        mm,
        grid=(_N // 500,),
        in_specs=[
            pl.BlockSpec((500, _D), lambda i: (i, 0)),
            pl.BlockSpec((_D, _D), lambda i: (0, 0)),
        ],
        out_specs=pl.BlockSpec((500, _D), lambda i: (i, 0)),
        out_shape=jax.ShapeDtypeStruct((_N, _D), jnp.float32),
    )(x, W)


def _add_relu(p):
    def ar(p_ref, o_ref):
        o_ref[...] = jnp.maximum(p_ref[0] + p_ref[1], 0.0)

    return pl.pallas_call(
        ar,
        grid=(_N // 500,),
        in_specs=[pl.BlockSpec((2, 500, _D), lambda i: (0, i, 0))],
        out_specs=pl.BlockSpec((500, _D), lambda i: (i, 0)),
        out_shape=jax.ShapeDtypeStruct((_N, _D), jnp.float32),
    )(p)


def _sc_aggregate(h, src, dst, w):
    mesh = plsc.VectorSubcoreMesh(core_axis_name="c", subcore_axis_name="s")

    @functools.partial(
        pl.kernel,
        out_type=jax.ShapeDtypeStruct((2, _N, _D), jnp.float32),
        mesh=mesh,
        scratch_types=[
            pltpu.VMEM((_NCHUNK, _CH), jnp.int32),
            pltpu.VMEM((_NCHUNK, _CH), jnp.int32),
            pltpu.VMEM((_EPT,), jnp.float32),
            pltpu.VMEM((_CH, _D), jnp.float32),
            pltpu.VMEM((_ZR, _D), jnp.float32),
            pltpu.VMEM_SHARED((_N, _D), jnp.float32),
            pltpu.SemaphoreType.DMA,
        ],
    )
    def k(h_hbm, src_hbm, dst_hbm, w_hbm, out_hbm,
          src_v, dst_v, w_v, rows_v, z_v, acc, sem):
        c = lax.axis_index("c")
        s = lax.axis_index("s")
        wid = c * 16 + s

        pltpu.sync_copy(src_hbm.at[wid], src_v)
        pltpu.sync_copy(dst_hbm.at[wid], dst_v)
        pltpu.sync_copy(w_hbm.at[wid], w_v)

        zero16 = jnp.zeros((16,), jnp.float32)

        def zfill(j, carry):
            for d_ in range(_D // 16):
                z_v[j, pl.ds(d_ * 16, 16)] = zero16
            return carry

        lax.fori_loop(0, _ZR, zfill, 0)

        def zcopy(i, carry):
            pltpu.sync_copy(z_v, acc.at[pl.ds(s * _RPS + i * _ZR, _ZR)])
            return carry

        lax.fori_loop(0, _RPS // _ZR, zcopy, 0)

        plsc.subcore_barrier()

        def chunk(i, carry):
            pltpu.async_copy(h_hbm.at[src_v.at[i]], rows_v, sem).wait()
            base = i * _CH

            def scale(j, c2):
                wvec = plsc.load_gather(
                    w_v, [jnp.full((16,), base + j, jnp.int32)])
                for d_ in range(_D // 16):
                    sl = pl.ds(d_ * 16, 16)
                    rows_v[j, sl] = rows_v[j, sl] * wvec
                return c2

            lax.fori_loop(0, _CH, scale, 0)
            pltpu.sync_copy(rows_v, acc.at[dst_v.at[i]], add=True)
            return carry

        lax.fori_loop(0, _NCHUNK, chunk, 0)

        plsc.subcore_barrier()

        def wb(i, carry):
            r0 = s * _RPS + i * _ZR
            pltpu.sync_copy(acc.at[pl.ds(r0, _ZR)], z_v)
            pltpu.sync_copy(z_v, out_hbm.at[c, pl.ds(r0, _ZR)])
            return carry

        lax.fori_loop(0, _RPS // _ZR, wb, 0)

    return k(h, src, dst, w)


def kernel(x, edge_index, edge_weight, W):
    h = _matmul(x, W)
    pad = _EPAD - _E
    src = jnp.pad(edge_index[1], (0, pad)).reshape(_TILES, _NCHUNK, _CH)
    dst = jnp.pad(edge_index[0], (0, pad)).reshape(_TILES, _NCHUNK, _CH)
    w = jnp.pad(edge_weight, (0, pad)).reshape(_TILES, _EPT)
    partial = _sc_aggregate(h, src, dst, w)
    return _add_relu(partial)


# speedup vs baseline: 3.1785x; 3.1785x over previous
import functools

import jax
import jax.numpy as jnp
from jax import lax
from jax.experimental import pallas as pl
from jax.experimental.pallas import tpu as pltpu
from jax.experimental.pallas import tpu_sc as plsc

_N = 10000
_E = 320000
_D = 128
_F = _D // 2

_CH = 128
_NCHUNK = 157
_EPT = _CH * _NCHUNK
_EPAD = 16 * _EPT
_WB = 200
_NWB = _N // _WB


def _matmul(x, W):
    def mm(x_ref, w_ref, o_ref):
        o_ref[...] = jnp.dot(x_ref[...], w_ref[...],
                             preferred_element_type=jnp.float32)

    return pl.pallas_call(
        mm,
        grid=(_N // 400,),
        in_specs=[
            pl.BlockSpec((400, _D), lambda i: (i, 0)),
            pl.BlockSpec((_D, _D), lambda i: (0, 0)),
        ],
        out_specs=pl.BlockSpec((400, _D), lambda i: (i, 0)),
        out_shape=jax.ShapeDtypeStruct((_N, _D), jnp.float32),
    )(x, W)


def _sc_aggregate(h2, src2, dst, wexp):
    mesh = plsc.VectorSubcoreMesh(core_axis_name="c", subcore_axis_name="s")

    @functools.partial(
        pl.kernel,
        out_type=jax.ShapeDtypeStruct((_N, 2, _F), jnp.float32),
        mesh=mesh,
        scratch_types=[
            pltpu.VMEM((_NCHUNK, _CH), jnp.int32),
            pltpu.VMEM((_NCHUNK, _CH), jnp.int32),
            pltpu.VMEM((_CH, 16), jnp.float32),
            pltpu.VMEM((_CH, _F), jnp.float32),
            pltpu.VMEM((_WB, _F), jnp.float32),
            pltpu.VMEM_SHARED((_N, _F), jnp.float32),
            pltpu.SemaphoreType.DMA,
        ],
        compiler_params=pltpu.CompilerParams(use_tc_tiling_on_sc=False),
    )
    def k(h_hbm, src_hbm, dst_hbm, w_hbm, out_hbm,
          src_v, dst_v, wexp_v, rows_v, z_v, acc, sem):
        c = lax.axis_index("c")
        s = lax.axis_index("s")

        pltpu.sync_copy(src_hbm.at[c, s], src_v)
        pltpu.sync_copy(dst_hbm.at[s], dst_v)

        zero16 = jnp.zeros((16,), jnp.float32)
        n_my_chunks = (_NWB - s + 15) // 16

        def zfill(j, carry):
            for d_ in range(_F // 16):
                z_v[j, pl.ds(d_ * 16, 16)] = zero16
            return carry

        lax.fori_loop(0, _WB, zfill, 0)

        def zcopy(i, carry):
            pltpu.sync_copy(z_v, acc.at[pl.ds((s + 16 * i) * _WB, _WB)])
            return carry

        lax.fori_loop(0, n_my_chunks, zcopy, 0)

        plsc.subcore_barrier()

        def chunk(i, carry):
            cp = pltpu.async_copy(h_hbm.at[src_v.at[i]], rows_v, sem)
            pltpu.sync_copy(w_hbm.at[s, i], wexp_v)
            cp.wait()

            def scale(j, c2):
                wvec = wexp_v[j]
                for d_ in range(_F // 16):
                    sl = pl.ds(d_ * 16, 16)
                    rows_v[j, sl] = rows_v[j, sl] * wvec
                return c2

            lax.fori_loop(0, _CH, scale, 0)
            pltpu.sync_copy(rows_v, acc.at[dst_v.at[i]], add=True)
            return carry

        lax.fori_loop(0, _NCHUNK, chunk, 0)

        plsc.subcore_barrier()

        def wb(i, carry):
            r0 = (s + 16 * i) * _WB
            pltpu.sync_copy(acc.at[pl.ds(r0, _WB)], z_v)

            def rl(j, c2):
                for d_ in range(_F // 16):
                    sl = pl.ds(d_ * 16, 16)
                    z_v[j, sl] = jnp.maximum(z_v[j, sl], 0.0)
                return c2

            lax.fori_loop(0, _WB, rl, 0)
            pltpu.sync_copy(z_v, out_hbm.at[pl.ds(r0, _WB), c])
            return carry

        lax.fori_loop(0, n_my_chunks, wb, 0)

    return k(h2, src2, dst, wexp)


def kernel(x, edge_index, edge_weight, W):
    h = _matmul(x, W)
    h2 = h.reshape(2 * _N, _F)
    pad = _EPAD - _E
    src = jnp.pad(edge_index[1], (0, pad))
    src2 = jnp.stack([src * 2, src * 2 + 1]).reshape(2, 16, _NCHUNK, _CH)
    dst = jnp.pad(edge_index[0], (0, pad)).reshape(16, _NCHUNK, _CH)
    wexp = jnp.broadcast_to(
        jnp.pad(edge_weight, (0, pad))[:, None], (_EPAD, 16)
    ).reshape(16, _NCHUNK, _CH, 16)
    out = _sc_aggregate(h2, src2, dst, wexp)
    return out.reshape(_N, _D)
